# baseline (speedup 1.0000x reference)
import functools

import jax
import jax.numpy as jnp
from jax.experimental import pallas as pl
from jax.experimental.pallas import tpu as pltpu

_NUM_NOUN = 42
_NUM_VERB = 4
_CH = 256
_LATENT = 256
_NV1, _NV2, _NV3 = 1723, 431, 108
_EPS = 1e-5
_F32 = jnp.float32
_BF = jnp.bfloat16


def _col(a):
    return a.reshape(-1, 1)


def _hl(a):
    hi = a.astype(_BF)
    lo = (a - hi.astype(_F32)).astype(_BF)
    return jnp.stack([hi, lo])


def _split(y):
    yh = y.astype(_BF)
    yl = (y - yh.astype(_F32)).astype(_BF)
    return yh, yl


def _mm(a, b):
    return jnp.dot(a, b, preferred_element_type=_F32)


def _dot3l(whl, y):
    yh, yl = _split(y)
    return _mm(whl[0], yh) + _mm(whl[0], yl) + _mm(whl[1], yh)


def _dot3r(x, whl):
    xh, xl = _split(x)
    return _mm(xh, whl[0]) + _mm(xl, whl[0]) + _mm(xh, whl[1])


def _pick(m, b):
    lane = jax.lax.broadcasted_iota(jnp.int32, m.shape, 1)
    return jnp.sum(jnp.where(lane == b, m, 0.0), axis=1, keepdims=True)


def _gn(y, g, b):
    C, V = y.shape
    s = jnp.sum(y, axis=1, keepdims=True)
    q = jnp.sum(y * y, axis=1, keepdims=True)
    t = jnp.concatenate([s, q], axis=1)
    for sh in (1, 2, 4):
        t = t + jnp.roll(t, -sh, axis=0)
    row = jax.lax.broadcasted_iota(jnp.int32, t.shape, 0)
    t = jnp.where(row % 8 == 0, t, 0.0)
    for sh in (1, 2, 4):
        t = t + jnp.roll(t, sh, axis=0)
    inv = 1.0 / (8.0 * V)
    m = t[:, 0:1] * inv
    var = t[:, 1:2] * inv - m * m
    a = jax.lax.rsqrt(var + _EPS) * g
    return y * a + (b - m * a)


def _grb(x, A_T, w, pre):
    g = lambda n: w[pre + n]
    y = jnp.maximum(_gn(x, g("preg"), g("preb")), 0.0)
    y = _dot3l(g("w1"), y) + g("b1")
    y = jnp.maximum(_gn(y, g("n1g"), g("n1b")), 0.0)
    y = _dot3r(_dot3l(g("wc"), y), A_T) + g("bc")
    y = jnp.maximum(_gn(y, g("n2g"), g("n2b")), 0.0)
    y = _dot3l(g("w2"), y) + g("b2")
    if (pre + "ws") in w:
        x = _dot3l(g("ws"), x) + g("bs")
    return x + y


def _grb_flat(dst, pre, p):
    dst[pre + "preg"] = _col(p["pre_norm"]["g"])
    dst[pre + "preb"] = _col(p["pre_norm"]["b"])
    dst[pre + "w1"] = _hl(p["lin1"]["W"])
    dst[pre + "b1"] = _col(p["lin1"]["b"])
    dst[pre + "n1g"] = _col(p["norm1"]["g"])
    dst[pre + "n1b"] = _col(p["norm1"]["b"])
    dst[pre + "wc"] = _hl(p["conv"]["W"])
    dst[pre + "bc"] = _col(p["conv"]["b"])
    dst[pre + "n2g"] = _col(p["norm2"]["g"])
    dst[pre + "n2b"] = _col(p["norm2"]["b"])
    dst[pre + "w2"] = _hl(p["lin2"]["W"])
    dst[pre + "b2"] = _col(p["lin2"]["b"])
    if "skip" in p:
        dst[pre + "ws"] = _hl(p["skip"]["W"])
        dst[pre + "bs"] = _col(p["skip"]["b"])


def _const_map(ndim):
    return lambda b: (0,) * ndim


def _enc_body(names, xc_ref, ic_ref, *rest):
    wrefs, out_ref = rest[:-1], rest[-1]
    w = {n: r[...] for n, r in zip(names, wrefs)}
    b = pl.program_id(0)
    icp = _dot3l(w["Wic"], ic_ref[...])
    y = _dot3l(w["Wxc"], xc_ref[0]) + _pick(icp, b) + w["b0"]
    y = _grb(y, w["A1T"], w, "e0_")
    y = _grb(y, w["A1T"], w, "e1_")
    y = _dot3r(y, w["D1T"])
    y = _grb(y, w["A2T"], w, "e2_")
    y = _grb(y, w["A2T"], w, "e3_")
    y = _dot3r(y, w["D2T"])
    y = _grb(y, w["A3T"], w, "e4_")
    y = _grb(y, w["A3T"], w, "e5_")
    out_ref[0] = y


def _enc_call(xcT, icT, w, Bn):
    names = tuple(sorted(w))
    ws = [w[n] for n in names]
    return pl.pallas_call(
        functools.partial(_enc_body, names),
        grid=(Bn,),
        in_specs=[pl.BlockSpec((1,) + xcT.shape[1:], lambda b: (b, 0, 0)),
                  pl.BlockSpec(icT.shape, _const_map(2))]
        + [pl.BlockSpec(a.shape, _const_map(a.ndim)) for a in ws],
        out_specs=pl.BlockSpec((1, _CH, _NV3), lambda b: (b, 0, 0)),
        out_shape=jax.ShapeDtypeStruct((Bn, _CH, _NV3), _F32),
        compiler_params=pltpu.CompilerParams(
            dimension_semantics=("parallel",),
            vmem_limit_bytes=100 * 1024 * 1024,
        ),
    )(xcT, icT, *ws)


def _lat_body(f_ref, w_ref, b_ref, o_ref):
    o_ref[...] = _dot3l(w_ref[...], f_ref[...]) + b_ref[...]


def _lat_call(featT, Wl, blc):
    return pl.pallas_call(
        _lat_body,
        out_shape=jax.ShapeDtypeStruct((_LATENT, featT.shape[1]), _F32),
        compiler_params=pltpu.CompilerParams(
            vmem_limit_bytes=100 * 1024 * 1024,
        ),
    )(featT, Wl, blc)


def _dec_body(names, z_ref, ic_ref, *rest):
    wrefs, ox_ref, of_ref = rest[:-2], rest[-2], rest[-1]
    w = {n: r[...] for n, r in zip(names, wrefs)}
    b = pl.program_id(0)
    zp = _dot3l(w["Wz"], z_ref[...])
    icp = _dot3l(w["Wicd"], ic_ref[...])
    y = _dot3l(w["Wrefd"], w["refb"]) + _pick(zp, b) + _pick(icp, b) + w["bd"]
    y = _grb(y, w["A3T"], w, "d0_")
    y = _grb(y, w["A3T"], w, "d1_")
    y = _dot3r(y, w["U2T"])
    y = _grb(y, w["A2T"], w, "d2_")
    y = _grb(y, w["A2T"], w, "d3_")
    y = _dot3r(y, w["U1T"])
    y = _grb(y, w["A1T"], w, "d4_")
    y = _grb(y, w["A1T"], w, "d5_")
    y = _grb(y, w["A1T"], w, "ga_")
    y = _grb(y, w["A1T"], w, "gb_")
    y = jnp.maximum(_gn(y, w["fing"], w["finb"]), 0.0)
    px = _dot3l(w["Wo1"], y) + w["bo1"]
    pf = _dot3l(w["Wo2"], y) + w["bo2"]
    icr = _dot3l(w["Wicr"], ic_ref[...])
    r = _dot3l(w["Wrr"], w["rinit"]) + _pick(icr, b) + w["br0"]
    r = jnp.maximum(r * w["bn0g"] + w["bn0b"], 0.0)
    for blk in ("r0", "r1"):
        t = _dot3l(w[blk + "c1w"], r) + w[blk + "c1b"]
        t = jnp.maximum(t * w[blk + "bn1g"] + w[blk + "bn1b"], 0.0)
        t = _dot3l(w[blk + "c2w"], t) + w[blk + "c2b"]
        t = t * w[blk + "bn2g"] + w[blk + "bn2b"]
        r = jnp.maximum(r + t, 0.0)
    pf = pf + _dot3l(w["Wor"], r) + w["bor"]
    row = jax.lax.broadcasted_iota(jnp.int32, pf.shape, 0)
    f = jnp.where(row == 0, 1.0 / (1.0 + jnp.exp(-pf)), pf)
    ox_ref[0] = px
    of_ref[0] = f


def _dec_call(zc, icT, w, Bn):
    names = tuple(sorted(w))
    ws = [w[n] for n in names]
    return pl.pallas_call(
        functools.partial(_dec_body, names),
        grid=(Bn,),
        in_specs=[pl.BlockSpec(zc.shape, _const_map(2)),
                  pl.BlockSpec(icT.shape, _const_map(2))]
        + [pl.BlockSpec(a.shape, _const_map(a.ndim)) for a in ws],
        out_specs=[pl.BlockSpec((1, 3, _NV1), lambda b: (b, 0, 0)),
                   pl.BlockSpec((1, _NUM_NOUN + 1, _NV1), lambda b: (b, 0, 0))],
        out_shape=[jax.ShapeDtypeStruct((Bn, 3, _NV1), _F32),
                   jax.ShapeDtypeStruct((Bn, _NUM_NOUN + 1, _NV1), _F32)],
        compiler_params=pltpu.CompilerParams(
            dimension_semantics=("parallel",),
            vmem_limit_bytes=100 * 1024 * 1024,
        ),
    )(zc, icT, *ws)


def kernel(body_vertices, contact_features, interaction_code, params, bufs):
    Bn = body_vertices.shape[0]
    bnscale = 1.0 / jnp.sqrt(jnp.float32(1.0 + _EPS))

    xcT = jnp.concatenate([body_vertices, contact_features],
                          axis=2).transpose(0, 2, 1)
    icT = interaction_code.T

    ew = {
        "A1T": _hl(bufs["A1"].T), "A2T": _hl(bufs["A2"].T),
        "A3T": _hl(bufs["A3"].T),
        "D1T": _hl(bufs["D1"].T), "D2T": _hl(bufs["D2"].T),
        "Wxc": _hl(params["enc_gl"]["W"][:, :46]),
        "Wic": _hl(params["enc_gl"]["W"][:, 46:]),
        "b0": _col(params["enc_gl"]["b"]),
    }
    for i, p in enumerate(params["enc_grb"]):
        _grb_flat(ew, f"e{i}_", p)

    y3 = _enc_call(xcT, icT, ew, Bn)

    featT = y3.reshape(Bn, _CH * _NV3).T
    zc = _lat_call(featT, _hl(params["latent"]["W"]),
                   _col(params["latent"]["b"]))

    Wd = params["dec_gl"]["W"]
    dw = {
        "A1T": ew["A1T"], "A2T": ew["A2T"], "A3T": ew["A3T"],
        "U1T": _hl(bufs["U1"].T), "U2T": _hl(bufs["U2"].T),
        "refb": bufs["ref"],
        "rinit": bufs["ref_init"],
        "Wz": _hl(Wd[:, :_LATENT]),
        "Wicd": _hl(Wd[:, _LATENT:_LATENT + _NUM_VERB * _NUM_NOUN]),
        "Wrefd": _hl(Wd[:, _LATENT + _NUM_VERB * _NUM_NOUN:]),
        "bd": _col(params["dec_gl"]["b"]),
        "fing": _col(params["gn_final"]["g"]),
        "finb": _col(params["gn_final"]["b"]),
        "Wo1": _hl(params["out_gl"]["W"][:3]),
        "bo1": _col(params["out_gl"]["b"][:3]),
        "Wo2": _hl(params["out_gl"]["W"][3:]),
        "bo2": _col(params["out_gl"]["b"][3:]),
    }
    for i, p in enumerate(params["dec_grb"]):
        _grb_flat(dw, f"d{i}_", p)
    _grb_flat(dw, "ga_", params["grb_a"])
    _grb_flat(dw, "gb_", params["grb_b"])

    res = params["res"]
    Wf = res["fc0"]["W"]
    dw.update({
        "Wicr": _hl(Wf[:, :_NUM_VERB * _NUM_NOUN]),
        "Wrr": _hl(Wf[:, _NUM_VERB * _NUM_NOUN:]),
        "br0": _col(res["fc0"]["b"]),
        "bn0g": _col(res["bn0"]["g"]) * bnscale,
        "bn0b": _col(res["bn0"]["b"]),
        "Wor": _hl(res["out"]["W"]),
        "bor": _col(res["out"]["b"]),
    })
    for j, blk in enumerate(res["blocks"]):
        pre = f"r{j}"
        dw.update({
            pre + "c1w": _hl(blk["c1"]["W"]), pre + "c1b": _col(blk["c1"]["b"]),
            pre + "bn1g": _col(blk["bn1"]["g"]) * bnscale,
            pre + "bn1b": _col(blk["bn1"]["b"]),
            pre + "c2w": _hl(blk["c2"]["W"]), pre + "c2b": _col(blk["c2"]["b"]),
            pre + "bn2g": _col(blk["bn2"]["g"]) * bnscale,
            pre + "bn2b": _col(blk["bn2"]["b"]),
        })

    pxT, fT = _dec_call(zc, icT, dw, Bn)
    return pxT.transpose(0, 2, 1), fT.transpose(0, 2, 1)
